# baseline (speedup 1.0000x reference)
import functools

import jax
import jax.numpy as jnp
import numpy as np
from jax.experimental import pallas as pl
from jax.experimental.pallas import tpu as pltpu

_N_BINS = 20


def _ece_stage1(x_ref, lab_ref, out_ref, cnt_ref, asum_ref, csum_ref, *,
                nsteps):
    j = pl.program_id(0)

    @pl.when(j == 0)
    def _init():
        cnt_ref[...] = jnp.zeros_like(cnt_ref)
        asum_ref[...] = jnp.zeros_like(asum_ref)
        csum_ref[...] = jnp.zeros_like(csum_ref)

    x = x_ref[0]
    G, E, C = x.shape
    R = G * E
    lab = lab_ref[0]

    m = jnp.max(x, axis=2)
    s = jnp.sum(jnp.exp(x - m[:, :, None]), axis=2)
    lanes = jax.lax.broadcasted_iota(jnp.int32, (G, E, C), 2)
    xl = jnp.max(jnp.where(lanes == lab[:, :, None], x, -jnp.inf), axis=2)
    conf = 1.0 / s
    phi = jnp.where(xl == m, -conf, conf)

    pr = phi.T[None]
    conf_row = jnp.abs(pr)
    acc_row = (pr < 0.0).astype(jnp.float32)
    bounds = (jax.lax.broadcasted_iota(jnp.int32, (_N_BINS, 1, 1), 0)
              ).astype(jnp.float32) / np.float32(_N_BINS)
    g = (conf_row > bounds).astype(jnp.float32)
    gshift = jnp.concatenate(
        [g[1:], jnp.zeros((1, E, G), jnp.float32)], axis=0)
    onehot = g - gshift

    cnt_ref[...] += onehot
    asum_ref[...] += onehot * acc_row
    csum_ref[...] += onehot * conf_row

    @pl.when(j == nsteps - 1)
    def _fin():
        cnt = jnp.sum(cnt_ref[...], axis=(1, 2))[:, None]
        asum = jnp.sum(asum_ref[...], axis=(1, 2))[:, None]
        csum = jnp.sum(csum_ref[...], axis=(1, 2))[:, None]
        out_ref[...] = jnp.concatenate([cnt, asum, csum], axis=1)


def _ece_stage2(p_ref, o_ref, *, n_total):
    tot = p_ref[...]
    cnt = tot[:, 0:1]
    asum = tot[:, 1:2]
    csum = tot[:, 2:3]
    prop = cnt / np.float32(n_total)
    denom = jnp.maximum(cnt, 1.0)
    contrib = jnp.where(cnt > 0.0,
                        jnp.abs(csum / denom - asum / denom) * prop,
                        0.0)
    o_ref[...] = jnp.sum(contrib, axis=0, keepdims=True)


def kernel(logits, labels):
    n, c = logits.shape
    rows = 10000
    nsteps = n // rows
    grp = rows // 8
    x4 = logits.reshape(nsteps, grp, 8, c)
    lab3 = labels.reshape(nsteps, grp, 8)

    parts = pl.pallas_call(
        functools.partial(_ece_stage1, nsteps=nsteps),
        grid=(nsteps,),
        in_specs=[
            pl.BlockSpec((1, grp, 8, c), lambda j: (j, 0, 0, 0)),
            pl.BlockSpec((1, grp, 8), lambda j: (j, 0, 0)),
        ],
        out_specs=pl.BlockSpec((_N_BINS, 3), lambda j: (0, 0)),
        out_shape=jax.ShapeDtypeStruct((_N_BINS, 3), jnp.float32),
        scratch_shapes=[
            pltpu.VMEM((_N_BINS, 8, rows // 8), jnp.float32),
            pltpu.VMEM((_N_BINS, 8, rows // 8), jnp.float32),
            pltpu.VMEM((_N_BINS, 8, rows // 8), jnp.float32),
        ],
    )(x4, lab3)

    out = pl.pallas_call(
        functools.partial(_ece_stage2, n_total=n),
        out_shape=jax.ShapeDtypeStruct((1, 1), jnp.float32),
    )(parts)
    return out.reshape(1)
